# baseline (speedup 1.0000x reference)
import jax
import jax.numpy as jnp
from jax import lax
from jax.experimental import pallas as pl
from jax.experimental.pallas import tpu as pltpu
from jax.experimental.pallas import tpu_sc as plsc

_TOPK = 3
_MARGIN = 100.0
_LAMBDA_SPARSITY = 0.008
_LAMBDA_SMOOTH = 0.0008
_HARD_NEG_RATIO = 0.3

_I32_MIN = -2147483648
_I32_MAGN = 2147483647

_NC = 2
_NS = 16
_LANES = 16

_B = 128
_T = 8192
_ROWS_PER_W = _B // (_NC * _NS)
_VREGS = _T // _LANES
_UNROLL = 8


def _row_top3_sum(row_ref):
    neg_inf = jnp.full((_LANES,), -jnp.inf, jnp.float32)

    def body(j, carry):
        m1, m2, m3 = carry
        for u in range(_UNROLL):
            v = row_ref[pl.ds((j * _UNROLL + u) * _LANES, _LANES)]
            a = jnp.minimum(m1, v)
            m1 = jnp.maximum(m1, v)
            b = jnp.minimum(m2, a)
            m2 = jnp.maximum(m2, a)
            m3 = jnp.maximum(m3, b)
        return m1, m2, m3

    m1, m2, m3 = lax.fori_loop(0, _VREGS // _UNROLL, body,
                               (neg_inf, neg_inf, neg_inf))
    t1 = jnp.float32(-jnp.inf)
    t2 = jnp.float32(-jnp.inf)
    t3 = jnp.float32(-jnp.inf)
    for vec in (m1, m2, m3):
        for l in range(_LANES):
            v = vec[l]
            a = jnp.minimum(t1, v)
            t1 = jnp.maximum(t1, v)
            b = jnp.minimum(t2, a)
            t2 = jnp.maximum(t2, a)
            t3 = jnp.maximum(t3, b)
    return t1 + t2 + t3


def _sc_top3_body(anom_hbm, out_hbm, row_v, vals_v):
    wid = lax.axis_index("s") * _NC + lax.axis_index("c")
    lane = lax.iota(jnp.int32, _LANES)

    vals = jnp.zeros((_LANES,), jnp.float32)
    for i in range(_ROWS_PER_W):
        r = wid * _ROWS_PER_W + i
        pltpu.sync_copy(anom_hbm.at[r], row_v)
        s3 = _row_top3_sum(row_v)
        vals = jnp.where(lane == i, s3, vals)

    vals_v[...] = vals
    pltpu.sync_copy(vals_v, out_hbm.at[wid])


def _sc_top3(anom):
    mesh = plsc.VectorSubcoreMesh(core_axis_name="c", subcore_axis_name="s",
                                  num_cores=_NC, num_subcores=_NS)
    fn = pl.kernel(
        _sc_top3_body,
        out_type=jax.ShapeDtypeStruct((_NC * _NS, _LANES), jnp.float32),
        mesh=mesh,
        scratch_types=[
            pltpu.VMEM((_T,), jnp.float32),
            pltpu.VMEM((_LANES,), jnp.float32),
        ],
    )
    return fn(anom)


def _unkey(s):
    i = s ^ (lax.shift_right_arithmetic(s, 31) & jnp.int32(_I32_MAGN))
    return lax.bitcast_convert_type(i, jnp.float32)


def _topk_mean(x, k):
    rows = x.shape[0]
    kf = jnp.float32(k)

    def body(it, p):
        j = 31 - it
        bit = lax.shift_left(jnp.int32(1), j)
        c = p | bit
        tf = _unkey(c ^ jnp.int32(_I32_MIN))
        cnt = jnp.sum(jnp.where(x >= tf, 1.0, 0.0), axis=1, keepdims=True)
        return jnp.where(cnt >= kf, c, p)

    p = lax.fori_loop(0, 32, body, jnp.zeros((rows, 1), jnp.int32))
    tf = _unkey(p ^ jnp.int32(_I32_MIN))
    gt = x > tf
    n_gt = jnp.sum(jnp.where(gt, 1.0, 0.0), axis=1, keepdims=True)
    sum_gt = jnp.sum(jnp.where(gt, x, 0.0), axis=1, keepdims=True)
    return (sum_gt + (kf - n_gt) * tf) / kf


def _tc_main_body(anom_ref, norm_ref, nmean_ref, sp_ref, sm_ref):
    anom = anom_ref[...]
    norm = norm_ref[...]
    b_a, t_a = anom.shape
    b_n, t_n = norm.shape
    hard_k = max(1, int(t_n * _HARD_NEG_RATIO))

    nmean_ref[...] = _topk_mean(norm, hard_k)

    sp_ref[0, 0] = (jnp.sum(anom) / jnp.float32(b_a * t_a)
                    + jnp.sum(norm) / jnp.float32(b_n * t_n)) * 0.5

    diff_a = anom[:, 1:] - anom[:, :-1]
    diff_n = norm[:, 1:] - norm[:, :-1]
    sm_ref[0, 0] = (jnp.sum(diff_a * diff_a) / jnp.float32(b_a * (t_a - 1))
                    + jnp.sum(diff_n * diff_n)
                    / jnp.float32(b_n * (t_n - 1))) * 0.5


def _tc_main(anom, norm):
    scalar = jax.ShapeDtypeStruct((1, 1), jnp.float32)
    smem = pl.BlockSpec(memory_space=pltpu.SMEM)
    return pl.pallas_call(
        _tc_main_body,
        out_shape=(jax.ShapeDtypeStruct((_B, 1), jnp.float32), scalar,
                   scalar),
        in_specs=[pl.BlockSpec(memory_space=pltpu.VMEM)] * 2,
        out_specs=(pl.BlockSpec(memory_space=pltpu.VMEM), smem, smem),
    )(anom, norm)


def _tc_final_body(asum_ref, nmean_ref, sp_ref, sm_ref,
                   total_ref, rank_ref, osp_ref, osm_ref):
    a_mean = asum_ref[...] / jnp.float32(_TOPK)
    n_mean = nmean_ref[...]

    ones_col = jnp.ones((_B, 1), jnp.float32)
    norm_row = lax.dot_general(
        ones_col, n_mean,
        dimension_numbers=(((1,), (1,)), ((), ())),
        preferred_element_type=jnp.float32,
    )
    pairs = jnp.maximum(_MARGIN - a_mean + norm_row, 0.0)
    rank_loss = jnp.sum(pairs) / jnp.float32(_B * _B)

    sparsity = sp_ref[0, 0]
    smooth = sm_ref[0, 0]
    total_ref[0, 0] = rank_loss + _LAMBDA_SPARSITY * sparsity \
        + _LAMBDA_SMOOTH * smooth
    rank_ref[0, 0] = rank_loss
    osp_ref[0, 0] = sparsity
    osm_ref[0, 0] = smooth


def _tc_final(a_sum3, n_mean, sp, sm):
    scalar = jax.ShapeDtypeStruct((1, 1), jnp.float32)
    smem = pl.BlockSpec(memory_space=pltpu.SMEM)
    vmem = pl.BlockSpec(memory_space=pltpu.VMEM)
    return pl.pallas_call(
        _tc_final_body,
        out_shape=(scalar, scalar, scalar, scalar),
        in_specs=(vmem, vmem, smem, smem),
        out_specs=(smem, smem, smem, smem),
    )(a_sum3, n_mean, sp, sm)


def kernel(anom_scores, norm_scores):
    a_sums = _sc_top3(anom_scores)
    n_mean, sp, sm = _tc_main(anom_scores, norm_scores)
    a_sum3 = a_sums[:, :_ROWS_PER_W].reshape(_B, 1)
    total, rank, osp, osm = _tc_final(a_sum3, n_mean, sp, sm)
    return (total[0, 0], rank[0, 0], osp[0, 0], osm[0, 0])


# speedup vs baseline: 1.5982x; 1.5982x over previous
import jax
import jax.numpy as jnp
from jax.experimental import pallas as pl
from jax.experimental.pallas import tpu as pltpu

_TOPK = 3
_MARGIN = 100.0
_LAMBDA_SPARSITY = 0.008
_LAMBDA_SMOOTH = 0.0008
_HARD_NEG_RATIO = 0.3

_I32_MIN = -2147483648
_I32_MAGN = 2147483647


def _unkey(s):
    i = s ^ (jax.lax.shift_right_arithmetic(s, 31) & jnp.int32(_I32_MAGN))
    return jax.lax.bitcast_convert_type(i, jnp.float32)


def _topk_mean(x, k):
    rows = x.shape[0]
    kf = jnp.float32(k)

    def body(it, p):
        j = 31 - it
        bit = jax.lax.shift_left(jnp.int32(1), j)
        c = p | bit
        tf = _unkey(c ^ jnp.int32(_I32_MIN))
        cnt = jnp.sum(jnp.where(x >= tf, 1, 0).astype(jnp.int32),
                      axis=1, keepdims=True)
        return jnp.where(cnt >= k, c, p)

    p = jax.lax.fori_loop(0, 32, body, jnp.zeros((rows, 1), jnp.int32))
    tf = _unkey(p ^ jnp.int32(_I32_MIN))
    gt = x > tf
    n_gt = jnp.sum(gt.astype(jnp.float32), axis=1, keepdims=True)
    sum_gt = jnp.sum(jnp.where(gt, x, 0.0), axis=1, keepdims=True)
    return (sum_gt + (kf - n_gt) * tf) / kf


def _top3_mean(x):
    neg_inf = jnp.float32(-jnp.inf)
    m1 = jnp.max(x, axis=1, keepdims=True)
    c1 = jnp.sum(jnp.where(x == m1, 1.0, 0.0), axis=1, keepdims=True)
    x2 = jnp.where(x < m1, x, neg_inf)
    m2 = jnp.max(x2, axis=1, keepdims=True)
    c2 = jnp.sum(jnp.where(x2 == m2, 1.0, 0.0), axis=1, keepdims=True)
    m3 = jnp.max(jnp.where(x2 < m2, x2, neg_inf), axis=1, keepdims=True)
    n1 = jnp.minimum(c1, 3.0)
    n2 = jnp.minimum(c2, 3.0 - n1)
    n3 = 3.0 - n1 - n2
    s = n1 * m1
    s = s + jnp.where(n2 > 0.0, n2 * m2, 0.0)
    s = s + jnp.where(n3 > 0.0, n3 * m3, 0.0)
    return s / 3.0


def _loss_kernel(anom_ref, norm_ref, total_ref, rank_ref, sp_ref, sm_ref):
    anom = anom_ref[...]
    norm = norm_ref[...]
    b_a, t_a = anom.shape
    b_n, t_n = norm.shape
    hard_k = max(1, int(t_n * _HARD_NEG_RATIO))

    anom_mean = _top3_mean(anom)
    norm_mean = _topk_mean(norm, hard_k)

    ones_col = jnp.ones((b_a, 1), jnp.float32)
    norm_row = jax.lax.dot_general(
        ones_col, norm_mean,
        dimension_numbers=(((1,), (1,)), ((), ())),
        preferred_element_type=jnp.float32,
    )
    pairs = jnp.maximum(_MARGIN - anom_mean + norm_row, 0.0)
    rank_loss = jnp.sum(pairs) / jnp.float32(b_a * b_n)

    sum_a = jnp.sum(anom)
    sum_n = jnp.sum(norm)
    sparsity = (sum_a / jnp.float32(b_a * t_a)
                + sum_n / jnp.float32(b_n * t_n)) * 0.5

    diff_a = anom[:, 1:] - anom[:, :-1]
    diff_n = norm[:, 1:] - norm[:, :-1]
    smooth = (jnp.sum(diff_a * diff_a) / jnp.float32(b_a * (t_a - 1))
              + jnp.sum(diff_n * diff_n) / jnp.float32(b_n * (t_n - 1))) * 0.5

    total_ref[0, 0] = rank_loss + _LAMBDA_SPARSITY * sparsity \
        + _LAMBDA_SMOOTH * smooth
    rank_ref[0, 0] = rank_loss
    sp_ref[0, 0] = sparsity
    sm_ref[0, 0] = smooth


def kernel(anom_scores, norm_scores):
    scalar = jax.ShapeDtypeStruct((1, 1), jnp.float32)
    smem = pl.BlockSpec(memory_space=pltpu.SMEM)
    total, rank, sp, sm = pl.pallas_call(
        _loss_kernel,
        out_shape=(scalar, scalar, scalar, scalar),
        in_specs=[pl.BlockSpec(memory_space=pltpu.VMEM)] * 2,
        out_specs=(smem, smem, smem, smem),
    )(anom_scores, norm_scores)
    return (total[0, 0], rank[0, 0], sp[0, 0], sm[0, 0])
